# baseline (speedup 1.0000x reference)
import jax
import jax.numpy as jnp
from jax import lax
from jax.experimental import pallas as pl
from jax.experimental.pallas import tpu as pltpu
from jax.experimental.pallas import tpu_sc as plsc

VOCAB = 1000000
EMBED_DIM = 32
BATCH = 4096
HIST_LEN = 200

NC = 2
NS = 16
NW = NC * NS
LANES = 16

GRAN = 16
NGRAN = VOCAB * EMBED_DIM // GRAN
BB = BATCH // NW
GH = 4
GROUPS = HIST_LEN // GH
GIDX = GH * BB * 2
PITCH = 129


CHUNK = 256
NCHUNK = VOCAB // CHUNK
TAIL = VOCAB - NCHUNK * CHUNK


def _make_compact():
    mesh = plsc.VectorSubcoreMesh(core_axis_name="c", subcore_axis_name="s")

    def body(table_hbm, out_hbm, in_v, stage_v, sem_i, sem_o):
        wid = lax.axis_index("s") * NC + lax.axis_index("c")

        def in_desc(j, slot, n):
            return pltpu.make_async_copy(
                table_hbm.at[pl.ds(j * CHUNK, n), :],
                in_v.at[slot, pl.ds(0, n)], sem_i)

        def out_desc(j, slot, n):
            return pltpu.make_async_copy(
                stage_v.at[slot, pl.ds(0, n // 4)],
                out_hbm.at[pl.ds(j * (CHUNK // 4), n // 4)], sem_o)

        def repack(slot, n):
            def rbody(q, carry):
                for t in range(4):
                    r = q * 4 + t
                    for half in range(2):
                        v = in_v[slot, r, pl.ds(half * LANES, LANES)]
                        stage_v[slot, q,
                                pl.ds(t * EMBED_DIM + half * LANES, LANES)] = v
                return carry

            lax.fori_loop(0, n // 4, rbody, 0, unroll=8)

        iters = (NCHUNK + NW - 1) // NW

        @pl.when(wid < NCHUNK)
        def _():
            in_desc(wid, 0, CHUNK).start()

        def chunk(i, carry):
            j = i * NW + wid
            slot = lax.rem(i, 2)

            @pl.when((i + 1) * NW + wid < NCHUNK)
            def _():
                in_desc((i + 1) * NW + wid, lax.rem(i + 1, 2), CHUNK).start()

            @pl.when(jnp.logical_and(i >= 2, (i - 2) * NW + wid < NCHUNK))
            def _():
                out_desc((i - 2) * NW + wid, slot, CHUNK).wait()

            @pl.when(j < NCHUNK)
            def _():
                in_desc(j, slot, CHUNK).wait()
                repack(slot, CHUNK)
                out_desc(j, slot, CHUNK).start()

            return carry

        lax.fori_loop(0, iters, chunk, 0)
        for i in (iters - 2, iters - 1):
            j = i * NW + wid

            @pl.when(j < NCHUNK)
            def _():
                out_desc(j, i % 2, CHUNK).wait()

        @pl.when(wid == 0)
        def _():
            in_desc(NCHUNK, 0, TAIL).start()
            in_desc(NCHUNK, 0, TAIL).wait()
            repack(0, TAIL)
            out_desc(NCHUNK, 0, TAIL).start()
            out_desc(NCHUNK, 0, TAIL).wait()

    kern = pl.kernel(
        body,
        out_type=jax.ShapeDtypeStruct(
            (VOCAB * EMBED_DIM // 128, 128), jnp.float32),
        mesh=mesh,
        scratch_types=[
            pltpu.VMEM((2, CHUNK, EMBED_DIM), jnp.float32),
            pltpu.VMEM((2, CHUNK // 4, 128), jnp.float32),
            pltpu.SemaphoreType.DMA,
            pltpu.SemaphoreType.DMA,
        ],
        compiler_params=pltpu.CompilerParams(
            use_tc_tiling_on_sc=True, needs_layout_passes=False),
    )
    return kern


_compact = _make_compact()


def _make_gather():
    mesh = plsc.VectorSubcoreMesh(core_axis_name="c", subcore_axis_name="s")

    def body(idx_hbm, table_hbm, out_hbm, idx_v, idx2_v, rows_v, rowst_v,
             sem_g, sem_o):
        wid = lax.axis_index("s") * NC + lax.axis_index("c")
        pltpu.sync_copy(idx_hbm.at[:, pl.ds(wid * BB, BB)], idx_v)

        lane = lax.iota(jnp.int32, LANES)
        dt_lo = lax.shift_right_logical(lane, 3)
        dt_hi = dt_lo + 2
        dr = lax.rem(lane, 8)

        def gather_desc(slot):
            return pltpu.make_async_copy(
                table_hbm.at[idx2_v.at[slot]], rows_v.at[slot], sem_g)

        def out_desc(g, hh, slot):
            return pltpu.make_async_copy(
                rowst_v.at[slot, hh, :, :, pl.ds(0, BB)],
                out_hbm.at[g * GH + hh, :, wid],
                sem_o,
            )

        def expand(g, slot):
            dst = idx2_v.at[slot]
            for hh in range(GH):
                h = g * GH + hh
                for c in range(BB // LANES):
                    v = idx_v[h, pl.ds(c * LANES, LANES)]
                    v2 = v + v
                    pos = (hh * 2 * BB + 2 * c * LANES) + 2 * lane
                    plsc.store_scatter(dst, [pos], v2)
                    plsc.store_scatter(dst, [pos + 1], v2 + 1)

        def transpose(g, slot):
            for hh in range(GH):
                tref = rowst_v.at[slot, hh]
                base = hh * 2 * BB

                def tbody(b, carry):
                    bvec = lane * 0 + b
                    v0 = rows_v[slot, base + 2 * b]
                    v1 = rows_v[slot, base + 2 * b + 1]
                    plsc.store_scatter(tref, [dt_lo, dr, bvec], v0)
                    plsc.store_scatter(tref, [dt_hi, dr, bvec], v1)
                    return carry

                lax.fori_loop(0, BB, tbody, 0, unroll=8)

        expand(0, 0)
        gather_desc(0).start()

        def group(g, carry):
            slot = lax.rem(g, 2)

            @pl.when(g + 1 < GROUPS)
            def _():
                nslot = lax.rem(g + 1, 2)
                expand(g + 1, nslot)
                gather_desc(nslot).start()

            gather_desc(slot).wait()

            @pl.when(g >= 2)
            def _():
                for hh in range(GH):
                    out_desc(g - 2, hh, slot).wait()

            transpose(g, slot)
            for hh in range(GH):
                out_desc(g, hh, slot).start()
            return carry

        lax.fori_loop(0, GROUPS, group, 0)
        for g in (GROUPS - 2, GROUPS - 1):
            for hh in range(GH):
                out_desc(g, hh, g % 2).wait()

    kern = pl.kernel(
        body,
        out_type=jax.ShapeDtypeStruct(
            (HIST_LEN, EMBED_DIM // 8, NW, 8, BB), jnp.float32),
        mesh=mesh,
        scratch_types=[
            pltpu.VMEM((HIST_LEN, BB), jnp.int32),
            pltpu.VMEM((2, GIDX), jnp.int32),
            pltpu.VMEM((2, GIDX, GRAN), jnp.float32),
            pltpu.VMEM((2, GH, EMBED_DIM // 8, 8, PITCH), jnp.float32),
            pltpu.SemaphoreType.DMA,
            pltpu.SemaphoreType.DMA,
        ],
        compiler_params=pltpu.CompilerParams(
            use_tc_tiling_on_sc=False, needs_layout_passes=False),
    )
    return kern


_gather = _make_gather()


def kernel(inputs, embedding_matrix):
    idx = inputs.astype(jnp.int32).T
    table = _compact(embedding_matrix).reshape(NGRAN, GRAN)
    out5 = _gather(idx, table)
    return out5.transpose(2, 4, 0, 1, 3).reshape(BATCH, HIST_LEN, EMBED_DIM)


# speedup vs baseline: 1.0156x; 1.0156x over previous
import jax
import jax.numpy as jnp
from jax import lax
from jax.experimental import pallas as pl
from jax.experimental.pallas import tpu as pltpu
from jax.experimental.pallas import tpu_sc as plsc

VOCAB = 1000000
EMBED_DIM = 32
BATCH = 4096
HIST_LEN = 200

NC = 2
NS = 16
NW = NC * NS
LANES = 16

GRAN = 16
NGRAN = VOCAB * EMBED_DIM // GRAN
BB = BATCH // NW
GH = 5
GROUPS = HIST_LEN // GH
GIDX = GH * BB * 2
PITCH = 129


def _make_gather():
    mesh = plsc.VectorSubcoreMesh(core_axis_name="c", subcore_axis_name="s")

    def body(idx_hbm, table_hbm, out_hbm, idx_v, idx2_v, rows_v, rowst_v,
             sem_g, sem_o):
        wid = lax.axis_index("s") * NC + lax.axis_index("c")
        pltpu.sync_copy(idx_hbm.at[:, pl.ds(wid * BB, BB)], idx_v)

        lane = lax.iota(jnp.int32, LANES)
        dt_lo = lax.shift_right_logical(lane, 3)
        dt_hi = dt_lo + 2
        dr = lax.rem(lane, 8)

        def gather_desc(slot):
            return pltpu.make_async_copy(
                table_hbm.at[idx2_v.at[slot]], rows_v.at[slot], sem_g)

        def out_desc(g, hh, slot):
            return pltpu.make_async_copy(
                rowst_v.at[slot, hh, :, :, pl.ds(0, BB)],
                out_hbm.at[g * GH + hh, :, wid],
                sem_o,
            )

        def expand(g, slot):
            dst = idx2_v.at[slot]
            for hh in range(GH):
                h = g * GH + hh
                for c in range(BB // LANES):
                    v = idx_v[h, pl.ds(c * LANES, LANES)]
                    v2 = v * 8
                    pos = (hh * 2 * BB + 2 * c * LANES) + 2 * lane
                    plsc.store_scatter(dst, [pos], v2)
                    plsc.store_scatter(dst, [pos + 1], v2 + 1)

        def transpose(g, slot):
            for hh in range(GH):
                tref = rowst_v.at[slot, hh]
                base = hh * 2 * BB

                def tbody(b, carry):
                    bvec = lane * 0 + b
                    v0 = rows_v[slot, base + 2 * b]
                    v1 = rows_v[slot, base + 2 * b + 1]
                    plsc.store_scatter(tref, [dt_lo, dr, bvec], v0)
                    plsc.store_scatter(tref, [dt_hi, dr, bvec], v1)
                    return carry

                lax.fori_loop(0, BB, tbody, 0, unroll=16)

        expand(0, 0)
        gather_desc(0).start()

        def group(g, carry):
            slot = lax.rem(g, 2)

            @pl.when(g + 1 < GROUPS)
            def _():
                nslot = lax.rem(g + 1, 2)
                expand(g + 1, nslot)
                gather_desc(nslot).start()

            gather_desc(slot).wait()

            @pl.when(g >= 2)
            def _():
                for hh in range(GH):
                    out_desc(g - 2, hh, slot).wait()

            transpose(g, slot)
            for hh in range(GH):
                out_desc(g, hh, slot).start()
            return carry

        lax.fori_loop(0, GROUPS, group, 0)
        for g in (GROUPS - 2, GROUPS - 1):
            for hh in range(GH):
                out_desc(g, hh, g % 2).wait()

    kern = pl.kernel(
        body,
        out_type=jax.ShapeDtypeStruct(
            (HIST_LEN, EMBED_DIM // 8, NW, 8, BB), jnp.float32),
        mesh=mesh,
        scratch_types=[
            pltpu.VMEM((HIST_LEN, BB), jnp.int32),
            pltpu.VMEM((2, GIDX), jnp.int32),
            pltpu.VMEM((2, GIDX, GRAN), jnp.float32),
            pltpu.VMEM((2, GH, EMBED_DIM // 8, 8, PITCH), jnp.float32),
            pltpu.SemaphoreType.DMA,
            pltpu.SemaphoreType.DMA,
        ],
        compiler_params=pltpu.CompilerParams(
            use_tc_tiling_on_sc=False, needs_layout_passes=False),
    )
    return kern


_gather = _make_gather()


def kernel(inputs, embedding_matrix):
    idx = inputs.astype(jnp.int32).T
    table = jnp.pad(embedding_matrix, ((0, 0), (0, 96))).reshape(8000000, GRAN)
    out5 = _gather(idx, table)
    return out5.transpose(2, 4, 0, 1, 3).reshape(BATCH, HIST_LEN, EMBED_DIM)


# speedup vs baseline: 1.0910x; 1.0743x over previous
import jax
import jax.numpy as jnp
from jax import lax
from jax.experimental import pallas as pl
from jax.experimental.pallas import tpu as pltpu
from jax.experimental.pallas import tpu_sc as plsc

VOCAB = 1000000
EMBED_DIM = 32
BATCH = 4096
HIST_LEN = 200

NC = 2
NS = 16
NW = NC * NS
LANES = 16

GRAN = 16
NGRAN = VOCAB * EMBED_DIM // GRAN
BB = BATCH // NW
GH = 4
GROUPS = HIST_LEN // GH
GIDX = GH * BB * 2
PITCH = 129


def _make_gather():
    mesh = plsc.VectorSubcoreMesh(core_axis_name="c", subcore_axis_name="s")

    def body(idx_hbm, table_hbm, out_hbm, idx_v, idx2_v, rows_v, rowst_v,
             sem_g, sem_o):
        wid = lax.axis_index("s") * NC + lax.axis_index("c")
        pltpu.sync_copy(idx_hbm.at[:, pl.ds(wid * BB, BB)], idx_v)

        lane = lax.iota(jnp.int32, LANES)
        dt_lo = lax.shift_right_logical(lane, 3)
        dt_hi = dt_lo + 2
        dr = lax.rem(lane, 8)

        def gather_desc(slot):
            return pltpu.make_async_copy(
                table_hbm.at[idx2_v.at[slot]], rows_v.at[slot], sem_g)

        def out_desc(g, hh, slot):
            return pltpu.make_async_copy(
                rowst_v.at[slot, hh, :, :, pl.ds(0, BB)],
                out_hbm.at[g * GH + hh, :, wid],
                sem_o,
            )

        def expand(g, slot):
            dst = idx2_v.at[slot]
            for hh in range(GH):
                h = g * GH + hh
                for c in range(BB // LANES):
                    v = idx_v[h, pl.ds(c * LANES, LANES)]
                    v2 = v * 8
                    pos = (hh * 2 * BB + 2 * c * LANES) + 2 * lane
                    plsc.store_scatter(dst, [pos], v2)
                    plsc.store_scatter(dst, [pos + 1], v2 + 1)

        def transpose(g, slot):
            for hh in range(GH):
                tref = rowst_v.at[slot, hh]
                base = hh * 2 * BB

                def tbody(b, carry):
                    bvec = lane * 0 + b
                    v0 = rows_v[slot, base + 2 * b]
                    v1 = rows_v[slot, base + 2 * b + 1]
                    plsc.store_scatter(tref, [dt_lo, dr, bvec], v0)
                    plsc.store_scatter(tref, [dt_hi, dr, bvec], v1)
                    return carry

                lax.fori_loop(0, BB, tbody, 0, unroll=8)

        expand(0, 0)
        gather_desc(0).start()

        def group(g, carry):
            slot = lax.rem(g, 2)

            @pl.when(g + 1 < GROUPS)
            def _():
                nslot = lax.rem(g + 1, 2)
                expand(g + 1, nslot)
                gather_desc(nslot).start()

            gather_desc(slot).wait()

            @pl.when(g >= 2)
            def _():
                for hh in range(GH):
                    out_desc(g - 2, hh, slot).wait()

            transpose(g, slot)
            for hh in range(GH):
                out_desc(g, hh, slot).start()
            return carry

        lax.fori_loop(0, GROUPS, group, 0)
        for g in (GROUPS - 2, GROUPS - 1):
            for hh in range(GH):
                out_desc(g, hh, g % 2).wait()

    kern = pl.kernel(
        body,
        out_type=jax.ShapeDtypeStruct(
            (HIST_LEN, EMBED_DIM // 8, NW, 8, BB), jnp.float32),
        mesh=mesh,
        scratch_types=[
            pltpu.VMEM((HIST_LEN, BB), jnp.int32),
            pltpu.VMEM((2, GIDX), jnp.int32),
            pltpu.VMEM((2, GIDX, GRAN), jnp.float32),
            pltpu.VMEM((2, GH, EMBED_DIM // 8, 8, PITCH), jnp.float32),
            pltpu.SemaphoreType.DMA,
            pltpu.SemaphoreType.DMA,
        ],
        compiler_params=pltpu.CompilerParams(
            use_tc_tiling_on_sc=False, needs_layout_passes=False),
    )
    return kern


_gather = _make_gather()


def kernel(inputs, embedding_matrix):
    idx = inputs.astype(jnp.int32).T
    table = jnp.pad(embedding_matrix, ((0, 0), (0, 96))).reshape(8000000, GRAN)
    out5 = _gather(idx, table)
    return out5.transpose(2, 4, 0, 1, 3).reshape(BATCH, HIST_LEN, EMBED_DIM)


# speedup vs baseline: 1.0912x; 1.0002x over previous
import jax
import jax.numpy as jnp
from jax import lax
from jax.experimental import pallas as pl
from jax.experimental.pallas import tpu as pltpu
from jax.experimental.pallas import tpu_sc as plsc

VOCAB = 1000000
EMBED_DIM = 32
BATCH = 4096
HIST_LEN = 200

NC = 2
NS = 16
NW = NC * NS
LANES = 16

GRAN = 16
NGRAN = VOCAB * 128 // GRAN
BB = BATCH // NW
GH = 4
GROUPS = HIST_LEN // GH
GIDX = GH * BB * 2
PITCH = 129


def _make_gather():
    mesh = plsc.VectorSubcoreMesh(core_axis_name="c", subcore_axis_name="s")

    def body(idx_hbm, table_hbm, out_hbm, idx_v, idx2_v, rows_v, rowst_v,
             sem_g, sem_o):
        wid = lax.axis_index("s") * NC + lax.axis_index("c")
        pltpu.sync_copy(idx_hbm.at[:, pl.ds(wid * BB, BB)], idx_v)

        lane = lax.iota(jnp.int32, LANES)
        dt_lo = lax.shift_right_logical(lane, 3)
        dt_hi = dt_lo + 2
        dr = lax.rem(lane, 8)

        def gather_desc(slot):
            return pltpu.make_async_copy(
                table_hbm.at[idx2_v.at[slot]], rows_v.at[slot], sem_g)

        def out_desc(g, hh, slot):
            return pltpu.make_async_copy(
                rowst_v.at[slot, hh, :, :, pl.ds(0, BB)],
                out_hbm.at[g * GH + hh, :, wid],
                sem_o,
            )

        def expand(g, slot):
            dst = idx2_v.at[slot]
            for hh in range(GH):
                h = g * GH + hh
                for c in range(BB // LANES):
                    v = idx_v[h, pl.ds(c * LANES, LANES)]
                    v2 = v * 8
                    pos = (hh * 2 * BB + 2 * c * LANES) + 2 * lane
                    plsc.store_scatter(dst, [pos], v2)
                    plsc.store_scatter(dst, [pos + 1], v2 + 1)

        def transpose(g, slot):
            for hh in range(GH):
                tref = rowst_v.at[slot, hh]
                base = hh * 2 * BB

                def tbody(b, carry):
                    bvec = lane * 0 + b
                    v0 = rows_v[slot, base + 2 * b]
                    v1 = rows_v[slot, base + 2 * b + 1]
                    plsc.store_scatter(tref, [dt_lo, dr, bvec], v0)
                    plsc.store_scatter(tref, [dt_hi, dr, bvec], v1)
                    return carry

                lax.fori_loop(0, BB, tbody, 0, unroll=8)

        expand(0, 0)
        gather_desc(0).start()

        def group(g, carry):
            slot = lax.rem(g, 2)

            @pl.when(g + 1 < GROUPS)
            def _():
                nslot = lax.rem(g + 1, 2)
                expand(g + 1, nslot)
                gather_desc(nslot).start()

            gather_desc(slot).wait()

            @pl.when(g >= 2)
            def _():
                for hh in range(GH):
                    out_desc(g - 2, hh, slot).wait()

            transpose(g, slot)
            for hh in range(GH):
                out_desc(g, hh, slot).start()
            return carry

        lax.fori_loop(0, GROUPS, group, 0)
        for g in (GROUPS - 2, GROUPS - 1):
            for hh in range(GH):
                out_desc(g, hh, g % 2).wait()

    kern = pl.kernel(
        body,
        out_type=jax.ShapeDtypeStruct(
            (HIST_LEN, EMBED_DIM // 8, NW, 8, BB), jnp.float32),
        mesh=mesh,
        scratch_types=[
            pltpu.VMEM((HIST_LEN, BB), jnp.int32),
            pltpu.VMEM((2, GIDX), jnp.int32),
            pltpu.VMEM((2, GIDX, GRAN), jnp.float32),
            pltpu.VMEM((2, GH, EMBED_DIM // 8, 8, PITCH), jnp.float32),
            pltpu.SemaphoreType.DMA,
            pltpu.SemaphoreType.DMA,
        ],
        compiler_params=pltpu.CompilerParams(
            use_tc_tiling_on_sc=False, needs_layout_passes=False),
    )
    return kern


_gather = _make_gather()


def kernel(inputs, embedding_matrix):
    idx = inputs.astype(jnp.int32).T
    table = jnp.pad(embedding_matrix, ((0, 0), (0, 96))).reshape(NGRAN, GRAN)
    out5 = _gather(idx, table)
    return out5.transpose(2, 4, 0, 1, 3).reshape(BATCH, HIST_LEN, EMBED_DIM)
